# baseline (speedup 1.0000x reference)
import functools

import jax
import jax.numpy as jnp
from jax import lax
from jax.experimental import pallas as pl
from jax.experimental.pallas import tpu as pltpu
from jax.experimental.pallas import tpu_sc as plsc

LANES = 16
NC = 2
NS = 16
NW = NC * NS


@functools.partial(jax.jit, static_argnums=(3, 4, 5))
def _embed_add(x, token_table, pe, S, D, B):
    CP = 32
    pos_per_w = S // NW
    n_p = pos_per_w // CP
    n_steps = n_p * B
    mesh = plsc.VectorSubcoreMesh(core_axis_name="c", subcore_axis_name="s")

    @functools.partial(
        pl.kernel,
        mesh=mesh,
        compiler_params=pltpu.CompilerParams(use_tc_tiling_on_sc=False),
        out_type=jax.ShapeDtypeStruct((B, S, D), jnp.float32),
        scratch_types=[
            pltpu.VMEM((B, pos_per_w), jnp.int32),
            pltpu.VMEM((2 * CP, D), jnp.float32),
            pltpu.VMEM((2 * CP, D), jnp.float32),
            pltpu.SemaphoreType.DMA,
            pltpu.SemaphoreType.DMA((2,)),
            pltpu.SemaphoreType.DMA((2,)),
            pltpu.SemaphoreType.DMA((2,)),
        ],
    )
    def k(x_hbm, table_hbm, pe_hbm, out_hbm,
          idx_v, rowsb, peb, isem, gsem, psem, ssem):
        wid = lax.axis_index("s") * NC + lax.axis_index("c")
        wpos = wid * pos_per_w

        pltpu.async_copy(x_hbm.at[:, pl.ds(wpos, pos_per_w)], idx_v,
                         isem).wait()

        def start_gather(g):
            p = g >> 2
            b = g & 3
            buf = g & 1
            idx_sl = idx_v.at[b, pl.ds(p * CP, CP)]
            pltpu.async_copy(table_hbm.at[idx_sl],
                             rowsb.at[pl.ds(buf * CP, CP)], gsem.at[buf])

        def start_pe(p):
            pb = p & 1
            pltpu.async_copy(pe_hbm.at[pl.ds(wpos + p * CP, CP)],
                             peb.at[pl.ds(pb * CP, CP)], psem.at[pb])

        def drain(sem_entry):
            pltpu.make_async_copy(pe_hbm.at[pl.ds(0, CP)],
                                  rowsb.at[pl.ds(0, CP)], sem_entry).wait()

        start_pe(0)
        start_gather(0)

        def body(g, carry):
            p = g >> 2
            b = g & 3
            buf = g & 1
            pb = p & 1

            @pl.when(g >= 1)
            def _():
                drain(ssem.at[(g + 1) & 1])

            @pl.when(g < n_steps - 1)
            def _():
                start_gather(g + 1)

            @pl.when(jnp.logical_and(b == 3, g < n_steps - 1))
            def _():
                start_pe(p + 1)

            drain(gsem.at[buf])

            @pl.when(b == 0)
            def _():
                drain(psem.at[pb])

            rbase = buf * CP
            pbase = pb * CP

            @plsc.parallel_loop(0, CP, unroll=2)
            def add_row(r):
                for v in range(D // LANES):
                    sl = pl.ds(v * LANES, LANES)
                    plsc.addupdate(rowsb.at[rbase + r, sl],
                                   peb[pbase + r, sl])

            pltpu.async_copy(rowsb.at[pl.ds(rbase, CP)],
                             out_hbm.at[b, pl.ds(wpos + p * CP, CP)],
                             ssem.at[buf])
            return carry

        lax.fori_loop(0, n_steps, body, 0)
        drain(ssem.at[(n_steps - 1) & 1])

    return k(x, token_table, pe)


def kernel(x, token_table, pe):
    B, S = x.shape
    D = token_table.shape[1]
    return _embed_add(x.astype(jnp.int32), token_table, pe, S, D, B)


# speedup vs baseline: 5.9608x; 5.9608x over previous
import functools

import jax
import jax.numpy as jnp
from jax import lax
from jax.experimental import pallas as pl
from jax.experimental.pallas import tpu as pltpu
from jax.experimental.pallas import tpu_sc as plsc

LANES = 16
NC = 2
NS = 16
NW = NC * NS


@functools.partial(jax.jit, static_argnums=(3, 4, 5))
def _embed_add(x, token_table, pe, S, D, B):
    CP = 32
    pos_per_w = S // NW
    n_p = pos_per_w // CP
    n_steps = n_p * B
    mesh = plsc.VectorSubcoreMesh(core_axis_name="c", subcore_axis_name="s")

    @functools.partial(
        pl.kernel,
        mesh=mesh,
        out_type=jax.ShapeDtypeStruct((B, S, D), jnp.float32),
        scratch_types=[
            pltpu.VMEM((B, pos_per_w), jnp.int32),
            pltpu.VMEM((2 * CP, D), jnp.float32),
            pltpu.VMEM((2 * CP, D), jnp.float32),
            pltpu.SemaphoreType.DMA,
            pltpu.SemaphoreType.DMA((2,)),
            pltpu.SemaphoreType.DMA((2,)),
            pltpu.SemaphoreType.DMA((2,)),
        ],
    )
    def k(x_hbm, table_hbm, pe_hbm, out_hbm,
          idx_v, rowsb, peb, isem, gsem, psem, ssem):
        wid = lax.axis_index("s") * NC + lax.axis_index("c")
        wpos = wid * pos_per_w

        pltpu.async_copy(x_hbm.at[:, pl.ds(wpos, pos_per_w)], idx_v,
                         isem).wait()

        def start_gather(g):
            p = g >> 2
            b = g & 3
            buf = g & 1
            idx_sl = idx_v.at[b, pl.ds(p * CP, CP)]
            pltpu.async_copy(table_hbm.at[idx_sl],
                             rowsb.at[pl.ds(buf * CP, CP)], gsem.at[buf])

        def start_pe(p):
            pb = p & 1
            pltpu.async_copy(pe_hbm.at[pl.ds(wpos + p * CP, CP)],
                             peb.at[pl.ds(pb * CP, CP)], psem.at[pb])

        def drain(sem_entry):
            pltpu.make_async_copy(pe_hbm.at[pl.ds(0, CP)],
                                  rowsb.at[pl.ds(0, CP)], sem_entry).wait()

        start_pe(0)
        start_gather(0)

        def body(g, carry):
            p = g >> 2
            b = g & 3
            buf = g & 1
            pb = p & 1

            @pl.when(g >= 1)
            def _():
                drain(ssem.at[(g + 1) & 1])

            @pl.when(g < n_steps - 1)
            def _():
                start_gather(g + 1)

            @pl.when(jnp.logical_and(b == 3, g < n_steps - 1))
            def _():
                start_pe(p + 1)

            drain(gsem.at[buf])

            @pl.when(b == 0)
            def _():
                drain(psem.at[pb])

            rbase = buf * CP
            pbase = pb * CP

            @plsc.parallel_loop(0, CP, unroll=2)
            def add_row(r):
                for v in range(D // LANES):
                    sl = pl.ds(v * LANES, LANES)
                    plsc.addupdate(rowsb.at[rbase + r, sl],
                                   peb[pbase + r, sl])

            pltpu.async_copy(rowsb.at[pl.ds(rbase, CP)],
                             out_hbm.at[b, pl.ds(wpos + p * CP, CP)],
                             ssem.at[buf])
            return carry

        lax.fori_loop(0, n_steps, body, 0)
        drain(ssem.at[(n_steps - 1) & 1])

    return k(x, token_table, pe)


def kernel(x, token_table, pe):
    B, S = x.shape
    D = token_table.shape[1]
    return _embed_add(x.astype(jnp.int32), token_table, pe, S, D, B)


# speedup vs baseline: 5.9626x; 1.0003x over previous
import functools

import jax
import jax.numpy as jnp
from jax import lax
from jax.experimental import pallas as pl
from jax.experimental.pallas import tpu as pltpu
from jax.experimental.pallas import tpu_sc as plsc

LANES = 16
NC = 2
NS = 16
NW = NC * NS


@functools.partial(jax.jit, static_argnums=(3, 4, 5))
def _embed_add(x, token_table, pe, S, D, B):
    CP = 32
    pos_per_w = S // NW
    n_p = pos_per_w // CP
    n_steps = n_p * B
    mesh = plsc.VectorSubcoreMesh(core_axis_name="c", subcore_axis_name="s")

    @functools.partial(
        pl.kernel,
        mesh=mesh,
        compiler_params=pltpu.CompilerParams(skip_device_barrier=True),
        out_type=jax.ShapeDtypeStruct((B, S, D), jnp.float32),
        scratch_types=[
            pltpu.VMEM((B, pos_per_w), jnp.int32),
            pltpu.VMEM((2 * CP, D), jnp.float32),
            pltpu.VMEM((2 * CP, D), jnp.float32),
            pltpu.SemaphoreType.DMA,
            pltpu.SemaphoreType.DMA((2,)),
            pltpu.SemaphoreType.DMA((2,)),
            pltpu.SemaphoreType.DMA((2,)),
        ],
    )
    def k(x_hbm, table_hbm, pe_hbm, out_hbm,
          idx_v, rowsb, peb, isem, gsem, psem, ssem):
        wid = lax.axis_index("s") * NC + lax.axis_index("c")
        wpos = wid * pos_per_w

        pltpu.async_copy(x_hbm.at[:, pl.ds(wpos, pos_per_w)], idx_v,
                         isem).wait()

        def start_gather(g):
            p = g >> 2
            b = g & 3
            buf = g & 1
            idx_sl = idx_v.at[b, pl.ds(p * CP, CP)]
            pltpu.async_copy(table_hbm.at[idx_sl],
                             rowsb.at[pl.ds(buf * CP, CP)], gsem.at[buf])

        def start_pe(p):
            pb = p & 1
            pltpu.async_copy(pe_hbm.at[pl.ds(wpos + p * CP, CP)],
                             peb.at[pl.ds(pb * CP, CP)], psem.at[pb])

        def drain(sem_entry):
            pltpu.make_async_copy(pe_hbm.at[pl.ds(0, CP)],
                                  rowsb.at[pl.ds(0, CP)], sem_entry).wait()

        start_pe(0)
        start_gather(0)

        def body(g, carry):
            p = g >> 2
            b = g & 3
            buf = g & 1
            pb = p & 1

            @pl.when(g >= 1)
            def _():
                drain(ssem.at[(g + 1) & 1])

            @pl.when(g < n_steps - 1)
            def _():
                start_gather(g + 1)

            @pl.when(jnp.logical_and(b == 3, g < n_steps - 1))
            def _():
                start_pe(p + 1)

            drain(gsem.at[buf])

            @pl.when(b == 0)
            def _():
                drain(psem.at[pb])

            rbase = buf * CP
            pbase = pb * CP

            @plsc.parallel_loop(0, CP, unroll=2)
            def add_row(r):
                for v in range(D // LANES):
                    sl = pl.ds(v * LANES, LANES)
                    plsc.addupdate(rowsb.at[rbase + r, sl],
                                   peb[pbase + r, sl])

            pltpu.async_copy(rowsb.at[pl.ds(rbase, CP)],
                             out_hbm.at[b, pl.ds(wpos + p * CP, CP)],
                             ssem.at[buf])
            return carry

        lax.fori_loop(0, n_steps, body, 0)
        drain(ssem.at[(n_steps - 1) & 1])

    return k(x, token_table, pe)


def kernel(x, token_table, pe):
    B, S = x.shape
    D = token_table.shape[1]
    return _embed_add(x.astype(jnp.int32), token_table, pe, S, D, B)
